# baseline (speedup 1.0000x reference)
import functools

import jax
import jax.numpy as jnp
from jax import lax
from jax.experimental import pallas as pl
from jax.experimental.pallas import tpu as pltpu
from jax.experimental.pallas import tpu_sc as plsc

GROUP = 4
K = 1024
J = 16
LOGVAR_MIN, LOGVAR_MAX = -30.0, 20.0

SEG = 2304

NW = 32


def _score_body(zp_ref, s_ref, idx_ref):
    zp = zp_ref[...]
    nb = zp.shape[1]
    mu = zp[:GROUP, :]
    lv = jnp.clip(zp[GROUP:, :], LOGVAR_MIN, LOGVAR_MAX)
    iv = jnp.exp(-lv)
    ft = jnp.concatenate([mu * iv, 0.5 * (1.0 - iv)], axis=0)
    s = s_ref[...]
    saug = jnp.concatenate([s, s * s], axis=1)
    def split3(x):
        x1 = x.astype(jnp.bfloat16)
        r1 = x - x1.astype(jnp.float32)
        x2 = r1.astype(jnp.bfloat16)
        x3 = (r1 - x2.astype(jnp.float32)).astype(jnp.bfloat16)
        return x1, x2, x3

    s1, s2, s3 = split3(saug)
    f1, f2, f3 = split3(ft)
    scat = jnp.concatenate([s1, s1, s2, s1, s3, s2], axis=1)
    fcat = jnp.concatenate([f1, f2, f1, f3, f1, f2], axis=0)
    scores = jax.lax.dot_general(
        scat, fcat, (((1,), (0,)), ((), ())),
        preferred_element_type=jnp.float32)
    m = jnp.max(scores, axis=0, keepdims=True)
    iot = lax.broadcasted_iota(jnp.int32, (K, nb), 0)
    idx_ref[0, 0, :] = jnp.min(jnp.where(scores >= m, iot, K), axis=0)


def _tc_indices(zt2, s, interpret=False):
    n_bl = zt2.shape[1]
    n = n_bl * J
    segs = n_bl // SEG
    return pl.pallas_call(
        _score_body,
        grid=(J * segs,),
        in_specs=[
            pl.BlockSpec((2 * GROUP, SEG), lambda i: (i // segs, i % segs)),
            pl.BlockSpec((K, GROUP), lambda i: (0, 0)),
        ],
        out_specs=pl.BlockSpec((1, 1, SEG), lambda i: (i, 0, 0)),
        out_shape=jax.ShapeDtypeStruct((n // SEG, 1, SEG), jnp.int32),
        interpret=interpret,
    )(zt2, s)


def _sc_gather(table_flat, idx_flat):
    n = idx_flat.shape[0]
    bpw = n // NW
    nvec = bpw // J
    mesh = plsc.VectorSubcoreMesh(core_axis_name="c", subcore_axis_name="s")

    @functools.partial(
        pl.kernel,
        mesh=mesh,
        out_type=jax.ShapeDtypeStruct((NW, bpw * GROUP), jnp.float32),
        scratch_types=[
            pltpu.VMEM((K * GROUP,), jnp.float32),
            pltpu.VMEM((bpw,), jnp.int32),
            pltpu.VMEM((bpw * GROUP,), jnp.float32),
        ],
        compiler_params=pltpu.CompilerParams(needs_layout_passes=False),
    )
    def gather_k(table_hbm, idx_hbm, out_hbm, tbl_v, idx_v, out_v):
        wid = lax.axis_index("s") * 2 + lax.axis_index("c")
        pltpu.sync_copy(table_hbm, tbl_v)
        pltpu.sync_copy(idx_hbm.at[pl.ds(wid * bpw, bpw)], idx_v)
        lane = lax.broadcasted_iota(jnp.int32, (J,), 0)

        def body(il, _):
            lin = idx_v[pl.ds(il * J, J)] * GROUP
            base = il * (GROUP * J) + lane
            for g in range(GROUP):
                vals = plsc.load_gather(tbl_v, [lin + g])
                plsc.store_scatter(out_v, [base + g * J], vals)
            return _

        lax.fori_loop(0, nvec, body, None)
        pltpu.sync_copy(out_v, out_hbm.at[wid])

    return gather_k(table_flat, idx_flat)


def kernel(z, prior_samples):
    z = z.astype(jnp.float32)
    b, l, c2 = z.shape
    c = c2 // 2
    n = b * l * J

    zt2 = (z.reshape(b * l, 2, GROUP, J)
           .transpose(3, 1, 2, 0).reshape(2 * GROUP * J, b * l))

    idx_jmaj = _tc_indices(zt2, prior_samples)
    idx2d = idx_jmaj.reshape(b * l, J)
    indices = idx2d.reshape(b, l, c // GROUP)

    flat = _sc_gather(prior_samples.reshape(-1),
                      idx2d.reshape(n))
    zhat = flat.reshape(b, l, c)
    return zhat, indices


# speedup vs baseline: 1.1269x; 1.1269x over previous
import functools

import jax
import jax.numpy as jnp
from jax import lax
from jax.experimental import pallas as pl
from jax.experimental.pallas import tpu as pltpu
from jax.experimental.pallas import tpu_sc as plsc

GROUP = 4
K = 1024
J = 16
LOGVAR_MIN, LOGVAR_MAX = -30.0, 20.0

SEG = 2304

NW = 32


def _split3(x):
    x1 = x.astype(jnp.bfloat16)
    r1 = x - x1.astype(jnp.float32)
    x2 = r1.astype(jnp.bfloat16)
    x3 = (r1 - x2.astype(jnp.float32)).astype(jnp.bfloat16)
    return x1, x2, x3


def _prep_body(s_ref, scat_ref):
    s = s_ref[...]
    saug = jnp.concatenate([s, s * s], axis=1)
    s1, s2, s3 = _split3(saug)
    scat_ref[...] = jnp.concatenate([s1, s1, s2, s1, s3, s2], axis=1)


def _prep_scat(s, interpret=False):
    return pl.pallas_call(
        _prep_body,
        out_shape=jax.ShapeDtypeStruct((K, 48), jnp.bfloat16),
        interpret=interpret,
    )(s)


def _score_body(zp_ref, scat_ref, idx_ref):
    zp = zp_ref[...]
    nb = zp.shape[1]
    mu = zp[:GROUP, :]
    lv = jnp.clip(zp[GROUP:, :], LOGVAR_MIN, LOGVAR_MAX)
    iv = jnp.exp(-lv)
    ft = jnp.concatenate([mu * iv, 0.5 * (1.0 - iv)], axis=0)
    f1, f2, f3 = _split3(ft)
    fcat = jnp.concatenate([f1, f2, f1, f3, f1, f2], axis=0)
    scores = jax.lax.dot_general(
        scat_ref[...], fcat, (((1,), (0,)), ((), ())),
        preferred_element_type=jnp.float32)
    m = jnp.max(scores, axis=0, keepdims=True)
    rev = (jnp.float32(K - 1) -
           lax.broadcasted_iota(jnp.int32, (K, nb), 0).astype(jnp.float32))
    r = jnp.max(jnp.where(scores >= m, rev, jnp.float32(-1.0)), axis=0)
    idx_ref[0, 0, :] = (jnp.float32(K - 1) - r).astype(jnp.int32)


def _tc_indices(zt2, s, interpret=False):
    n_bl = zt2.shape[1]
    n = n_bl * J
    segs = n_bl // SEG
    return pl.pallas_call(
        _score_body,
        grid=(J * segs,),
        in_specs=[
            pl.BlockSpec((2 * GROUP, SEG), lambda i: (i // segs, i % segs)),
            pl.BlockSpec((K, 48), lambda i: (0, 0)),
        ],
        out_specs=pl.BlockSpec((1, 1, SEG), lambda i: (i, 0, 0)),
        out_shape=jax.ShapeDtypeStruct((n // SEG, 1, SEG), jnp.int32),
        interpret=interpret,
    )(zt2, s)


def _sc_gather(table_flat, idx_flat):
    n = idx_flat.shape[0]
    bpw = n // NW
    nvec = bpw // J
    mesh = plsc.VectorSubcoreMesh(core_axis_name="c", subcore_axis_name="s")

    @functools.partial(
        pl.kernel,
        mesh=mesh,
        out_type=jax.ShapeDtypeStruct((NW, bpw * GROUP), jnp.float32),
        scratch_types=[
            pltpu.VMEM((K * GROUP,), jnp.float32),
            pltpu.VMEM((bpw,), jnp.int32),
            pltpu.VMEM((bpw * GROUP,), jnp.float32),
        ],
        compiler_params=pltpu.CompilerParams(needs_layout_passes=False),
    )
    def gather_k(table_hbm, idx_hbm, out_hbm, tbl_v, idx_v, out_v):
        wid = lax.axis_index("s") * 2 + lax.axis_index("c")
        pltpu.sync_copy(table_hbm, tbl_v)
        pltpu.sync_copy(idx_hbm.at[pl.ds(wid * bpw, bpw)], idx_v)
        lane = lax.broadcasted_iota(jnp.int32, (J,), 0)

        def body(il, _):
            lin = idx_v[pl.ds(il * J, J)] * GROUP
            base = il * (GROUP * J) + lane
            for g in range(GROUP):
                vals = plsc.load_gather(tbl_v, [lin + g])
                plsc.store_scatter(out_v, [base + g * J], vals)
            return _

        lax.fori_loop(0, nvec, body, None)
        pltpu.sync_copy(out_v, out_hbm.at[wid])

    return gather_k(table_flat, idx_flat)


def kernel(z, prior_samples):
    z = z.astype(jnp.float32)
    b, l, c2 = z.shape
    c = c2 // 2
    n = b * l * J

    zt2 = (z.reshape(b * l, 2, GROUP, J)
           .transpose(3, 1, 2, 0).reshape(2 * GROUP * J, b * l))

    scat = _prep_scat(prior_samples)
    idx_jmaj = _tc_indices(zt2, scat)
    idx2d = idx_jmaj.reshape(J, b * l).T
    indices = idx2d.reshape(b, l, c // GROUP)

    flat = _sc_gather(prior_samples.reshape(-1),
                      idx2d.reshape(n))
    zhat = flat.reshape(b, l, c)
    return zhat, indices


# speedup vs baseline: 1.1342x; 1.0065x over previous
import functools

import jax
import jax.numpy as jnp
from jax import lax
from jax.experimental import pallas as pl
from jax.experimental.pallas import tpu as pltpu
from jax.experimental.pallas import tpu_sc as plsc

GROUP = 4
K = 1024
J = 16
LOGVAR_MIN, LOGVAR_MAX = -30.0, 20.0

SEG = 4608

NW = 32


def _split3(x):
    x1 = x.astype(jnp.bfloat16)
    r1 = x - x1.astype(jnp.float32)
    x2 = r1.astype(jnp.bfloat16)
    x3 = (r1 - x2.astype(jnp.float32)).astype(jnp.bfloat16)
    return x1, x2, x3


def _prep_body(s_ref, scat_ref):
    s = s_ref[...]
    saug = jnp.concatenate([s, s * s], axis=1)
    s1, s2, s3 = _split3(saug)
    scat_ref[...] = jnp.concatenate([s1, s1, s2, s1, s3, s2], axis=1)


def _prep_scat(s, interpret=False):
    return pl.pallas_call(
        _prep_body,
        out_shape=jax.ShapeDtypeStruct((K, 48), jnp.bfloat16),
        interpret=interpret,
    )(s)


def _score_body(zp_ref, scat_ref, idx_ref):
    zp = zp_ref[...]
    nb = zp.shape[1]
    mu = zp[:GROUP, :]
    lv = jnp.clip(zp[GROUP:, :], LOGVAR_MIN, LOGVAR_MAX)
    iv = jnp.exp(-lv)
    ft = jnp.concatenate([mu * iv, 0.5 * (1.0 - iv)], axis=0)
    f1, f2, f3 = _split3(ft)
    fcat = jnp.concatenate([f1, f2, f1, f3, f1, f2], axis=0)
    scores = jax.lax.dot_general(
        scat_ref[...], fcat, (((1,), (0,)), ((), ())),
        preferred_element_type=jnp.float32)
    m = jnp.max(scores, axis=0, keepdims=True)
    rev = (jnp.float32(K - 1) -
           lax.broadcasted_iota(jnp.int32, (K, nb), 0).astype(jnp.float32))
    r = jnp.max(jnp.where(scores >= m, rev, jnp.float32(-1.0)), axis=0)
    idx_ref[0, 0, :] = (jnp.float32(K - 1) - r).astype(jnp.int32)


def _tc_indices(zt2, s, interpret=False):
    n_bl = zt2.shape[1]
    n = n_bl * J
    segs = n_bl // SEG
    return pl.pallas_call(
        _score_body,
        grid=(J * segs,),
        in_specs=[
            pl.BlockSpec((2 * GROUP, SEG), lambda i: (i // segs, i % segs)),
            pl.BlockSpec((K, 48), lambda i: (0, 0)),
        ],
        out_specs=pl.BlockSpec((1, 1, SEG), lambda i: (i, 0, 0)),
        out_shape=jax.ShapeDtypeStruct((n // SEG, 1, SEG), jnp.int32),
        interpret=interpret,
    )(zt2, s)


def _sc_gather(table_flat, idx_flat):
    n = idx_flat.shape[0]
    bpw = n // NW
    nvec = bpw // J
    mesh = plsc.VectorSubcoreMesh(core_axis_name="c", subcore_axis_name="s")

    @functools.partial(
        pl.kernel,
        mesh=mesh,
        out_type=jax.ShapeDtypeStruct((NW, bpw * GROUP), jnp.float32),
        scratch_types=[
            pltpu.VMEM((K * GROUP,), jnp.float32),
            pltpu.VMEM((bpw,), jnp.int32),
            pltpu.VMEM((bpw * GROUP,), jnp.float32),
        ],
        compiler_params=pltpu.CompilerParams(needs_layout_passes=False),
    )
    def gather_k(table_hbm, idx_hbm, out_hbm, tbl_v, idx_v, out_v):
        wid = lax.axis_index("s") * 2 + lax.axis_index("c")
        pltpu.sync_copy(table_hbm, tbl_v)
        pltpu.sync_copy(idx_hbm.at[pl.ds(wid * bpw, bpw)], idx_v)
        lane = lax.broadcasted_iota(jnp.int32, (J,), 0)

        def body(il, _):
            lin = idx_v[pl.ds(il * J, J)] * GROUP
            base = il * (GROUP * J) + lane
            for g in range(GROUP):
                vals = plsc.load_gather(tbl_v, [lin + g])
                plsc.store_scatter(out_v, [base + g * J], vals)
            return _

        lax.fori_loop(0, nvec, body, None, unroll=4)
        pltpu.sync_copy(out_v, out_hbm.at[wid])

    return gather_k(table_flat, idx_flat)


def kernel(z, prior_samples):
    z = z.astype(jnp.float32)
    b, l, c2 = z.shape
    c = c2 // 2
    n = b * l * J

    zt2 = (z.reshape(b * l, 2, GROUP, J)
           .transpose(3, 1, 2, 0).reshape(2 * GROUP * J, b * l))

    scat = _prep_scat(prior_samples)
    idx_jmaj = _tc_indices(zt2, scat)
    idx2d = idx_jmaj.reshape(J, b * l).T
    indices = idx2d.reshape(b, l, c // GROUP)

    flat = _sc_gather(prior_samples.reshape(-1),
                      idx2d.reshape(n))
    zhat = flat.reshape(b, l, c)
    return zhat, indices
